# baseline (speedup 1.0000x reference)
import jax
import jax.numpy as jnp
from jax import lax
from jax.experimental import pallas as pl
from jax.experimental.pallas import tpu as pltpu
from jax.experimental.pallas import tpu_sc as plsc

NROW = 16384
NCOL = 64
NSUB = 16
NCORE = 2
COLS_PER_CORE = NCOL // NCORE
PER_TILE = NROW // NSUB
LANES = 16
NVREG = PER_TILE // LANES
UNROLL = 4

HALF = 500_000
DUMMY0 = 1_000_000
TBL_PAD = 1 << 20
ZBUF = 16384
ZPER_TILE = TBL_PAD // NSUB


def _scatter_body(out_hbm, idx_hbm, src_hbm,
                  idx_v, sa_v, ca_v, sb_v, cb_v, src_v, one_v, dum_v,
                  sum_v, cnt_v, val_v, flat_v, gidx_v, zrow_v, tbl, sem):
    cid = lax.axis_index("c")
    sid = lax.axis_index("s")

    def cfill(k, c):
        o = pl.ds(k * LANES, LANES)
        zrow_v[o] = jnp.zeros((LANES,), jnp.float32)
        return c
    lax.fori_loop(0, ZBUF // LANES, cfill, 0)

    def cfill2(k, c):
        o = pl.ds(k * LANES, LANES)
        iota = lax.broadcasted_iota(jnp.int32, (LANES,), 0)
        one_v[o] = jnp.full((LANES,), 1.0, jnp.float32)
        dum_v[o] = DUMMY0 + sid * PER_TILE + k * LANES + iota
        gidx_v[o] = (sid * PER_TILE + k * LANES + iota) * NCOL + cid * COLS_PER_CORE
        return c
    lax.fori_loop(0, NVREG, cfill2, 0)

    hs = []
    for z in range(ZPER_TILE // ZBUF):
        hs.append(pltpu.async_copy(
            zrow_v, tbl.at[pl.ds(sid * ZPER_TILE + z * ZBUF, ZBUF)], sem))
    for hh in hs:
        hh.wait()

    plsc.subcore_barrier()

    def col_body(jj, carry):
        col = cid * COLS_PER_CORE + jj
        h1 = pltpu.async_copy(idx_hbm.at[gidx_v], idx_v, sem)
        h2 = pltpu.async_copy(src_hbm.at[gidx_v], src_v, sem)
        h1.wait()
        h2.wait()

        def prep(k, c):
            for u in range(UNROLL):
                o = pl.ds((k * UNROLL + u) * LANES, LANES)
                v = idx_v[o]
                d = dum_v[o]
                ina = v < HALF
                sa_v[o] = jnp.where(ina, v, d)
                ca_v[o] = jnp.where(ina, v + HALF, d)
                sb_v[o] = jnp.where(ina, d, v - HALF)
                cb_v[o] = jnp.where(ina, d, v)
                flat_v[o] = v * NCOL + col
                val_v[o] = src_v[o]
                gidx_v[o] = gidx_v[o] + 1
            return c
        lax.fori_loop(0, NVREG // UNROLL, prep, 0)

        for h in range(2):
            sidx = sa_v if h == 0 else sb_v
            cidx = ca_v if h == 0 else cb_v

            h1 = pltpu.async_copy(src_v, tbl.at[sidx], sem, add=True)
            h2 = pltpu.async_copy(one_v, tbl.at[cidx], sem, add=True)
            h1.wait()
            h2.wait()
            plsc.subcore_barrier()

            h1 = pltpu.async_copy(tbl.at[sidx], sum_v, sem)
            h2 = pltpu.async_copy(tbl.at[cidx], cnt_v, sem)
            h1.wait()
            h2.wait()
            plsc.subcore_barrier()

            h1 = pltpu.async_copy(zrow_v.at[pl.ds(0, PER_TILE)], tbl.at[sidx], sem)
            h2 = pltpu.async_copy(zrow_v.at[pl.ds(0, PER_TILE)], tbl.at[cidx], sem)

            def mfill(k, c):
                for u in range(UNROLL):
                    o = pl.ds((k * UNROLL + u) * LANES, LANES)
                    v = idx_v[o]
                    inr = (v < HALF) if h == 0 else (v >= HALF)
                    mean = sum_v[o] / cnt_v[o]
                    val_v[o] = jnp.where(inr, mean, val_v[o])
                return c
            lax.fori_loop(0, NVREG // UNROLL, mfill, 0)

            h1.wait()
            h2.wait()
            plsc.subcore_barrier()

        pltpu.async_copy(val_v, out_hbm.at[flat_v], sem).wait()
        return carry

    lax.fori_loop(0, COLS_PER_CORE, col_body, 0)


_mesh = plsc.VectorSubcoreMesh(core_axis_name="c", subcore_axis_name="s")

_scatter = pl.kernel(
    _scatter_body,
    out_type=(),
    mesh=_mesh,
    scratch_types=[
        pltpu.VMEM((PER_TILE,), jnp.int32),
        pltpu.VMEM((PER_TILE,), jnp.int32),
        pltpu.VMEM((PER_TILE,), jnp.int32),
        pltpu.VMEM((PER_TILE,), jnp.int32),
        pltpu.VMEM((PER_TILE,), jnp.int32),
        pltpu.VMEM((PER_TILE,), jnp.float32),
        pltpu.VMEM((PER_TILE,), jnp.float32),
        pltpu.VMEM((PER_TILE,), jnp.int32),
        pltpu.VMEM((PER_TILE,), jnp.float32),
        pltpu.VMEM((PER_TILE,), jnp.float32),
        pltpu.VMEM((PER_TILE,), jnp.float32),
        pltpu.VMEM((PER_TILE,), jnp.int32),
        pltpu.VMEM((PER_TILE,), jnp.int32),
        pltpu.VMEM((ZBUF,), jnp.float32),
        pltpu.VMEM_SHARED((TBL_PAD,), jnp.float32),
        pltpu.SemaphoreType.DMA,
    ],
)


def kernel(x, index, src):
    out_ref = jax.new_ref(x.reshape(-1))
    _scatter(out_ref, index.reshape(-1), src.reshape(-1))
    return out_ref[...].reshape(x.shape)


# speedup vs baseline: 1.0093x; 1.0093x over previous
import jax
import jax.numpy as jnp
from jax import lax
from jax.experimental import pallas as pl
from jax.experimental.pallas import tpu as pltpu
from jax.experimental.pallas import tpu_sc as plsc

NROW = 16384
NCOL = 64
NSUB = 16
NCORE = 2
COLS_PER_CORE = NCOL // NCORE
PER_TILE = NROW // NSUB
LANES = 16
NVREG = PER_TILE // LANES

HALF = 500_000
DUMMY0 = 1_000_000
TBL_PAD = 1 << 20
ZCHUNK = 1024
ZPER_TILE = TBL_PAD // NSUB


def _scatter_body(out_hbm, idx_hbm, src_hbm,
                  idx_v, sidx_v, cidx_v, src_v, one_v, zer_v,
                  sum_v, cnt_v, val_v, flat_v, tbl, sem):
    cid = lax.axis_index("c")
    sid = lax.axis_index("s")

    def cfill(k, c):
        one_v[pl.ds(k * LANES, LANES)] = jnp.full((LANES,), 1.0, jnp.float32)
        zer_v[pl.ds(k * LANES, LANES)] = jnp.zeros((LANES,), jnp.float32)
        return c
    lax.fori_loop(0, NVREG, cfill, 0)

    def zinit(k, c):
        pltpu.sync_copy(zer_v, tbl.at[pl.ds(sid * ZPER_TILE + k * ZCHUNK, ZCHUNK)])
        return c
    lax.fori_loop(0, ZPER_TILE // ZCHUNK, zinit, 0)

    plsc.subcore_barrier()

    def col_body(jj, carry):
        col = cid * COLS_PER_CORE + jj
        pltpu.sync_copy(idx_hbm.at[col, pl.ds(sid * PER_TILE, PER_TILE)], idx_v)
        pltpu.sync_copy(src_hbm.at[col, pl.ds(sid * PER_TILE, PER_TILE)], src_v)

        def ffill(k, c):
            v = idx_v[pl.ds(k * LANES, LANES)]
            flat_v[pl.ds(k * LANES, LANES)] = v * NCOL + col
            val_v[pl.ds(k * LANES, LANES)] = src_v[pl.ds(k * LANES, LANES)]
            return c
        lax.fori_loop(0, NVREG, ffill, 0)

        def half_body(h, c0):
            base = h * HALF
            def sfill(k, c):
                v = idx_v[pl.ds(k * LANES, LANES)]
                dummy = (DUMMY0 + sid * PER_TILE + k * LANES
                         + lax.broadcasted_iota(jnp.int32, (LANES,), 0))
                inr = (v >= base) & (v < base + HALF)
                sidx_v[pl.ds(k * LANES, LANES)] = jnp.where(inr, v - base, dummy)
                cidx_v[pl.ds(k * LANES, LANES)] = jnp.where(inr, v - base + HALF, dummy)
                return c
            lax.fori_loop(0, NVREG, sfill, 0)

            h1 = pltpu.async_copy(src_v, tbl.at[sidx_v], sem, add=True)
            h2 = pltpu.async_copy(one_v, tbl.at[cidx_v], sem, add=True)
            h1.wait()
            h2.wait()
            plsc.subcore_barrier()

            h1 = pltpu.async_copy(tbl.at[sidx_v], sum_v, sem)
            h2 = pltpu.async_copy(tbl.at[cidx_v], cnt_v, sem)
            h1.wait()
            h2.wait()
            plsc.subcore_barrier()

            h1 = pltpu.async_copy(zer_v, tbl.at[sidx_v], sem)
            h2 = pltpu.async_copy(zer_v, tbl.at[cidx_v], sem)

            def mfill(k, c):
                v = idx_v[pl.ds(k * LANES, LANES)]
                inr = (v >= base) & (v < base + HALF)
                mean = sum_v[pl.ds(k * LANES, LANES)] / cnt_v[pl.ds(k * LANES, LANES)]
                val_v[pl.ds(k * LANES, LANES)] = jnp.where(
                    inr, mean, val_v[pl.ds(k * LANES, LANES)])
                return c
            lax.fori_loop(0, NVREG, mfill, 0)

            h1.wait()
            h2.wait()
            plsc.subcore_barrier()
            return c0

        lax.fori_loop(0, 2, half_body, 0)

        pltpu.async_copy(val_v, out_hbm.at[flat_v], sem).wait()
        return carry

    lax.fori_loop(0, COLS_PER_CORE, col_body, 0)


_mesh = plsc.VectorSubcoreMesh(core_axis_name="c", subcore_axis_name="s")

_scatter = pl.kernel(
    _scatter_body,
    out_type=(),
    mesh=_mesh,
    scratch_types=[
        pltpu.VMEM((PER_TILE,), jnp.int32),
        pltpu.VMEM((PER_TILE,), jnp.int32),
        pltpu.VMEM((PER_TILE,), jnp.int32),
        pltpu.VMEM((PER_TILE,), jnp.float32),
        pltpu.VMEM((PER_TILE,), jnp.float32),
        pltpu.VMEM((PER_TILE,), jnp.float32),
        pltpu.VMEM((PER_TILE,), jnp.float32),
        pltpu.VMEM((PER_TILE,), jnp.float32),
        pltpu.VMEM((PER_TILE,), jnp.float32),
        pltpu.VMEM((PER_TILE,), jnp.int32),
        pltpu.VMEM_SHARED((TBL_PAD,), jnp.float32),
        pltpu.SemaphoreType.DMA,
    ],
)


def kernel(x, index, src):
    idx_t = index.T
    src_t = src.T
    out_ref = jax.new_ref(x.reshape(-1))
    _scatter(out_ref, idx_t, src_t)
    return out_ref[...].reshape(x.shape)
